# Initial kernel scaffold; baseline (speedup 1.0000x reference)
import jax
import jax.numpy as jnp
from jax.experimental import pallas as pl


def kernel(x, edge_index, edge_attr, u, batch, W1, b1, W2, b2):
    raise NotImplementedError("write your pallas kernel here")


# speedup vs baseline: 6.4576x; 6.4576x over previous
import jax
import jax.numpy as jnp
from jax.experimental import pallas as pl
from jax.experimental.pallas import tpu as pltpu

_N = 100000
_D = 128
_G = 128
_B = 256
_RB = 1000
_NBLK = _N // _RB


def _tc_body(batch_ref, x_ref, u_ref, W1_ref, b1_ref, W2_ref, b2_ref,
             out_ref, acc, cnt):
    i = pl.program_id(0)

    @pl.when(i == 0)
    def _init():
        acc[...] = jnp.zeros_like(acc)
        cnt[...] = jnp.zeros_like(cnt)

    ids = batch_ref[0, 0, :]
    onehot = (jax.lax.broadcasted_iota(jnp.int32, (_B, _RB), 0)
              == ids[None, :]).astype(jnp.float32)
    acc[...] += jnp.dot(onehot, x_ref[...], preferred_element_type=jnp.float32)
    cnt[...] += jnp.sum(onehot, axis=1, keepdims=True)

    @pl.when(i == _NBLK - 1)
    def _finish():
        pooled = acc[...] / jnp.maximum(cnt[...], 1.0)
        h = jnp.maximum(
            jnp.dot(u_ref[...], W1_ref[0:_G, :],
                    preferred_element_type=jnp.float32)
            + jnp.dot(pooled, W1_ref[_G:_G + _D, :],
                      preferred_element_type=jnp.float32)
            + b1_ref[...], 0.0)
        out_ref[...] = (jnp.dot(h, W2_ref[...],
                                preferred_element_type=jnp.float32)
                        + b2_ref[...])


def kernel(x, edge_index, edge_attr, u, batch, W1, b1, W2, b2):
    del edge_index, edge_attr
    batch3d = batch.astype(jnp.int32).reshape(_NBLK, 1, _RB)
    return pl.pallas_call(
        _tc_body,
        grid=(_NBLK,),
        in_specs=[
            pl.BlockSpec((1, 1, _RB), lambda i: (i, 0, 0)),
            pl.BlockSpec((_RB, _D), lambda i: (i, 0)),
            pl.BlockSpec((_B, _G), lambda i: (0, 0)),
            pl.BlockSpec((_G + _D, _G), lambda i: (0, 0)),
            pl.BlockSpec((1, _G), lambda i: (0, 0)),
            pl.BlockSpec((_G, _G), lambda i: (0, 0)),
            pl.BlockSpec((1, _G), lambda i: (0, 0)),
        ],
        out_specs=pl.BlockSpec((_B, _G), lambda i: (0, 0)),
        out_shape=jax.ShapeDtypeStruct((_B, _G), jnp.float32),
        scratch_shapes=[pltpu.VMEM((_B, _G), jnp.float32),
                        pltpu.VMEM((_B, 1), jnp.float32)],
    )(batch3d, x, u, W1, b1.reshape(1, _G), W2, b2.reshape(1, _G))
